# baseline (speedup 1.0000x reference)
import jax
import jax.numpy as jnp
from jax import lax
from jax.experimental import pallas as pl
from jax.experimental.pallas import tpu as pltpu
from jax.experimental.pallas import tpu_sc as plsc

BATCH, SEQ, D = 4, 4096, 2048
NC, NS = 2, 16
NW = NC * NS
SEQ_PER_W = SEQ // NW
CS = 8
NCHUNK = SEQ_PER_W // CS
NBUF = 3
LANES = 16
VPR = D // LANES

SC_BATCH = 3
S_BLK = 256


def _tc_body(x_ref, emb_ref, out_ref):
    out_ref[0] = x_ref[0] + emb_ref[...]


def _sc_body(x_hbm, emb_hbm, out_hbm, xbuf, ebuf, insem, outsem):
    wid = lax.axis_index("s") * NC + lax.axis_index("c")
    s_base = wid * SEQ_PER_W

    def in_copies(ci, k):
        s0 = s_base + ci * CS
        return (
            pltpu.make_async_copy(
                x_hbm.at[SC_BATCH, pl.ds(s0, CS)], xbuf.at[k], insem.at[k]
            ),
            pltpu.make_async_copy(
                emb_hbm.at[pl.ds(s0, CS)], ebuf.at[k], insem.at[k]
            ),
        )

    def out_copy(ci, k):
        s0 = s_base + ci * CS
        return pltpu.make_async_copy(
            xbuf.at[k], out_hbm.at[0, pl.ds(s0, CS)], outsem.at[k]
        )

    def start_in(ci, k):
        for c in in_copies(ci, k):
            c.start()

    def wait_in(ci, k):
        for c in in_copies(ci, k):
            c.wait()

    def compute(k):
        @plsc.parallel_loop(0, VPR, step=1, unroll=4)
        def vec(j):
            off = j * LANES
            for s in range(CS):
                xbuf[k, s, pl.ds(off, LANES)] = (
                    xbuf[k, s, pl.ds(off, LANES)] + ebuf[k, s, pl.ds(off, LANES)]
                )

    start_in(0, 0)

    def step(ci, carry):
        k = lax.rem(ci, NBUF)
        kn = lax.rem(ci + 1, NBUF)

        @pl.when(jnp.logical_and(ci + 1 < NCHUNK, ci >= NBUF - 1))
        def _():
            out_copy(ci + 1 - NBUF, kn).wait()

        @pl.when(ci + 1 < NCHUNK)
        def _():
            start_in(ci + 1, kn)

        wait_in(ci, k)
        compute(k)
        out_copy(ci, k).start()
        return carry

    lax.fori_loop(0, NCHUNK, step, 0)
    for ci in range(NCHUNK - NBUF, NCHUNK):
        out_copy(ci, ci % NBUF).wait()


INTERPRET = False


def kernel(x, embedding):
    mesh = plsc.VectorSubcoreMesh(
        core_axis_name="c", subcore_axis_name="s", num_cores=NC, num_subcores=NS
    )
    sc_f = pl.kernel(
        _sc_body,
        jax.ShapeDtypeStruct((1, SEQ, D), jnp.float32),
        mesh=mesh,
        scratch_types=[
            pltpu.VMEM((NBUF, CS, D), jnp.float32),
            pltpu.VMEM((NBUF, CS, D), jnp.float32),
            pltpu.SemaphoreType.DMA((NBUF,)),
            pltpu.SemaphoreType.DMA((NBUF,)),
        ],
        interpret=INTERPRET,
    )
    sc_out = sc_f(x, embedding)

    n_seq = SEQ // S_BLK
    tc_out = pl.pallas_call(
        _tc_body,
        grid=(n_seq, SC_BATCH),
        in_specs=[
            pl.BlockSpec((1, S_BLK, D), lambda i, b: (b, i, 0)),
            pl.BlockSpec((S_BLK, D), lambda i, b: (i, 0)),
        ],
        out_specs=pl.BlockSpec((1, S_BLK, D), lambda i, b: (b, i, 0)),
        out_shape=jax.ShapeDtypeStruct((SC_BATCH, SEQ, D), jnp.float32),
    )(x, embedding)

    return jnp.concatenate([tc_out, sc_out], axis=0)


# speedup vs baseline: 1.7015x; 1.7015x over previous
import jax
import jax.numpy as jnp
from jax import lax
from jax.experimental import pallas as pl
from jax.experimental.pallas import tpu as pltpu
from jax.experimental.pallas import tpu_sc as plsc

BATCH, SEQ, D = 4, 4096, 2048
NC, NS = 2, 16
NW = NC * NS
SEQ_PER_W = SEQ // NW
CS = 4
NCHUNK = SEQ_PER_W // CS
NBUF = 3
LANES = 16
VPR = D // LANES


def _sc_body(x_hbm, emb_hbm, out_hbm, xbuf, ebuf, insem, outsem):
    wid = lax.axis_index("s") * NC + lax.axis_index("c")
    s_base = wid * SEQ_PER_W

    def in_copies(ci, k):
        s0 = s_base + ci * CS
        return (
            pltpu.make_async_copy(
                x_hbm.at[:, pl.ds(s0, CS)], xbuf.at[k], insem.at[k]
            ),
            pltpu.make_async_copy(
                emb_hbm.at[pl.ds(s0, CS)], ebuf.at[k], insem.at[k]
            ),
        )

    def out_copy(ci, k):
        s0 = s_base + ci * CS
        return pltpu.make_async_copy(
            xbuf.at[k], out_hbm.at[:, pl.ds(s0, CS)], outsem.at[k]
        )

    def start_in(ci, k):
        for c in in_copies(ci, k):
            c.start()

    def wait_in(ci, k):
        for c in in_copies(ci, k):
            c.wait()

    def compute(k):
        @plsc.parallel_loop(0, VPR, step=1, unroll=4)
        def vec(j):
            off = j * LANES
            for s in range(CS):
                e = ebuf[k, s, pl.ds(off, LANES)]
                for b in range(BATCH):
                    xbuf[k, b, s, pl.ds(off, LANES)] = (
                        xbuf[k, b, s, pl.ds(off, LANES)] + e
                    )

    start_in(0, 0)

    def step(ci, carry):
        k = lax.rem(ci, NBUF)
        kn = lax.rem(ci + 1, NBUF)

        @pl.when(jnp.logical_and(ci + 1 < NCHUNK, ci >= NBUF - 1))
        def _():
            out_copy(ci + 1 - NBUF, kn).wait()

        @pl.when(ci + 1 < NCHUNK)
        def _():
            start_in(ci + 1, kn)

        wait_in(ci, k)
        compute(k)
        out_copy(ci, k).start()
        return carry

    lax.fori_loop(0, NCHUNK, step, 0)
    for ci in range(NCHUNK - NBUF, NCHUNK):
        out_copy(ci, ci % NBUF).wait()


INTERPRET = False


def kernel(x, embedding):
    mesh = plsc.VectorSubcoreMesh(
        core_axis_name="c", subcore_axis_name="s", num_cores=NC, num_subcores=NS
    )
    f = pl.kernel(
        _sc_body,
        jax.ShapeDtypeStruct((BATCH, SEQ, D), jnp.float32),
        mesh=mesh,
        scratch_types=[
            pltpu.VMEM((NBUF, BATCH, CS, D), jnp.float32),
            pltpu.VMEM((NBUF, CS, D), jnp.float32),
            pltpu.SemaphoreType.DMA((NBUF,)),
            pltpu.SemaphoreType.DMA((NBUF,)),
        ],
        interpret=INTERPRET,
    )
    return f(x, embedding)
